# Initial kernel scaffold; baseline (speedup 1.0000x reference)
import jax
import jax.numpy as jnp
from jax.experimental import pallas as pl


def kernel(x_solvent, edge_index_solvent, edge_attr_solvent, x_solvent_batch, x_solute, edge_index_solute, edge_attr_solute, x_solute_batch, global_feat, num_graphs, sv_W1, sv_b1, sv_W2, sv_b2, sv_We, sv_be, sv_gamma, sv_beta, su_W1, su_b1, su_W2, su_b2, su_We, su_be, su_gamma, su_beta, fc_W, fc_b, out_W, out_b):
    raise NotImplementedError("write your pallas kernel here")


# speedup vs baseline: 1.9993x; 1.9993x over previous
import functools

import jax
import jax.numpy as jnp
from jax import lax
from jax.experimental import pallas as pl
from jax.experimental.pallas import tpu as pltpu
from jax.experimental.pallas import tpu_sc as plsc

N = 10000
E = 320000
D = 128
DE = 16
H = 128
G = 128
L = 3

NC = 2
NS = 16
NW = NC * NS

CHUNK = 128
NCHUNKS = E // CHUNK
CH_PER_TILE = -(-NCHUNKS // NW)

ZCH = 80
NZCH = N // ZCH
ZCH_PER_TILE = -(-NZCH // NS)

_SC_MESH = plsc.VectorSubcoreMesh(core_axis_name="c", subcore_axis_name="s")


@functools.partial(
    pl.kernel,
    out_type=jax.ShapeDtypeStruct((NC, N, D), jnp.float32),
    mesh=_SC_MESH,
    scratch_types=[
        pltpu.VMEM((CHUNK,), jnp.int32),
        pltpu.VMEM((CHUNK,), jnp.int32),
        pltpu.VMEM((CHUNK, D), jnp.float32),
        pltpu.VMEM((CHUNK, D), jnp.float32),
        pltpu.VMEM_SHARED((N, D), jnp.float32),
        pltpu.SemaphoreType.DMA,
    ],
)
def _edge_pass(x_hbm, e_hbm, src_hbm, dst_hbm, out_hbm,
               sidx, didx, ev, xv, agg, sem):
    cid = lax.axis_index("c")
    sid = lax.axis_index("s")
    wid = sid * NC + cid

    def _zrow(r, carry):
        for c in range(D // 16):
            ev[r, pl.ds(c * 16, 16)] = jnp.zeros((16,), jnp.float32)
        return carry

    lax.fori_loop(0, ZCH, _zrow, 0)
    for k in range(ZCH_PER_TILE):
        ch = k * NS + sid

        @pl.when(ch < NZCH)
        def _():
            pltpu.sync_copy(ev.at[pl.ds(0, ZCH)], agg.at[pl.ds(ch * ZCH, ZCH)])

    plsc.subcore_barrier()

    def _chunk(i, carry):
        ch = i * NW + wid

        @pl.when(ch < NCHUNKS)
        def _():
            base = ch * CHUNK
            pltpu.sync_copy(src_hbm.at[pl.ds(base, CHUNK)], sidx)
            pltpu.sync_copy(dst_hbm.at[pl.ds(base, CHUNK)], didx)
            pltpu.sync_copy(e_hbm.at[pl.ds(base, CHUNK)], ev)
            pltpu.async_copy(x_hbm.at[sidx], xv, sem).wait()

            def _row(r, c2):
                for c in range(D // 16):
                    s = pl.ds(c * 16, 16)
                    xv[r, s] = jnp.maximum(xv[r, s] + ev[r, s], 0.0)
                return c2

            lax.fori_loop(0, CHUNK, _row, 0)
            pltpu.sync_copy(xv, agg.at[didx], add=True)

        return carry

    lax.fori_loop(0, CH_PER_TILE, _chunk, 0)
    plsc.subcore_barrier()

    for k in range(ZCH_PER_TILE):
        ch = k * NS + sid

        @pl.when(ch < NZCH)
        def _():
            r0 = ch * ZCH
            pltpu.sync_copy(agg.at[pl.ds(r0, ZCH)], out_hbm.at[cid, pl.ds(r0, ZCH)])


PCH = 80
PNCH = N // PCH
PCH_PER_TILE = -(-PNCH // NW)
G_PER_TILE = G // NS


@functools.partial(
    pl.kernel,
    out_type=jax.ShapeDtypeStruct((NC, G, D), jnp.float32),
    mesh=_SC_MESH,
    scratch_types=[
        pltpu.VMEM((PCH,), jnp.int32),
        pltpu.VMEM((PCH, D), jnp.float32),
        pltpu.VMEM((G_PER_TILE, D), jnp.float32),
        pltpu.VMEM_SHARED((G, D), jnp.float32),
    ],
)
def _pool(h_hbm, b_hbm, out_hbm, bidx, hv, zv, acc):
    cid = lax.axis_index("c")
    sid = lax.axis_index("s")
    wid = sid * NC + cid

    for r in range(G_PER_TILE):
        for c in range(D // 16):
            zv[r, pl.ds(c * 16, 16)] = jnp.zeros((16,), jnp.float32)
    pltpu.sync_copy(zv, acc.at[pl.ds(sid * G_PER_TILE, G_PER_TILE)])
    plsc.subcore_barrier()

    def _chunk(i, carry):
        ch = i * NW + wid

        @pl.when(ch < PNCH)
        def _():
            base = ch * PCH
            pltpu.sync_copy(b_hbm.at[pl.ds(base, PCH)], bidx)
            pltpu.sync_copy(h_hbm.at[pl.ds(base, PCH)], hv)
            pltpu.sync_copy(hv, acc.at[bidx], add=True)

        return carry

    lax.fori_loop(0, PCH_PER_TILE, _chunk, 0)
    plsc.subcore_barrier()

    pltpu.sync_copy(acc.at[pl.ds(sid * G_PER_TILE, G_PER_TILE)],
                    out_hbm.at[cid, pl.ds(sid * G_PER_TILE, G_PER_TILE)])


BE = 1280


def _elin_body(ea_ref, we_ref, be_ref, o_ref):
    o_ref[...] = (
        jnp.dot(ea_ref[...], we_ref[...], preferred_element_type=jnp.float32)
        + be_ref[...]
    )


_elin = pl.pallas_call(
    _elin_body,
    grid=(E // BE,),
    in_specs=[
        pl.BlockSpec((BE, DE), lambda i: (i, 0)),
        pl.BlockSpec((DE, H), lambda i: (0, 0)),
        pl.BlockSpec((1, H), lambda i: (0, 0)),
    ],
    out_specs=pl.BlockSpec((BE, H), lambda i: (i, 0)),
    out_shape=jax.ShapeDtypeStruct((E, H), jnp.float32),
)

BN = 1000


def _node_body(x_ref, ap_ref, w1_ref, b1_ref, w2_ref, b2_ref, sc_ref, bt_ref,
               o_ref):
    h = x_ref[...] + ap_ref[0] + ap_ref[1]
    t = jnp.maximum(
        jnp.dot(h, w1_ref[...], preferred_element_type=jnp.float32) + b1_ref[...],
        0.0,
    )
    u = jnp.dot(t, w2_ref[...], preferred_element_type=jnp.float32) + b2_ref[...]
    o_ref[...] = jnp.maximum(u * sc_ref[...] + bt_ref[...], 0.0)


_node = pl.pallas_call(
    _node_body,
    grid=(N // BN,),
    in_specs=[
        pl.BlockSpec((BN, H), lambda i: (i, 0)),
        pl.BlockSpec((NC, BN, H), lambda i: (0, i, 0)),
        pl.BlockSpec((H, H), lambda i: (0, 0)),
        pl.BlockSpec((1, H), lambda i: (0, 0)),
        pl.BlockSpec((H, H), lambda i: (0, 0)),
        pl.BlockSpec((1, H), lambda i: (0, 0)),
        pl.BlockSpec((1, H), lambda i: (0, 0)),
        pl.BlockSpec((1, H), lambda i: (0, 0)),
    ],
    out_specs=pl.BlockSpec((BN, H), lambda i: (i, 0)),
    out_shape=jax.ShapeDtypeStruct((N, H), jnp.float32),
)


def _head_body(psv_ref, psu_ref, gf_ref, wa_ref, wb_ref, wg_ref, fb_ref,
               wo_ref, ob_ref, o_ref):
    gsv = psv_ref[0] + psv_ref[1]
    gsu = psu_ref[0] + psu_ref[1]
    h = (
        jnp.dot(gsv, wa_ref[...], preferred_element_type=jnp.float32)
        + jnp.dot(gsu, wb_ref[...], preferred_element_type=jnp.float32)
        + jnp.dot(gf_ref[...], wg_ref[...], preferred_element_type=jnp.float32)
        + fb_ref[...]
    )
    h = jnp.maximum(h, 0.0)
    o_ref[...] = jnp.dot(h, wo_ref[...], preferred_element_type=jnp.float32) + ob_ref[...]


_head = pl.pallas_call(
    _head_body,
    out_shape=jax.ShapeDtypeStruct((G, H), jnp.float32),
)


def _run_graph(x, edge_index, edge_attr, W1, b1, W2, b2, We, be, gamma, beta):
    src = edge_index[0]
    dst = edge_index[1]
    inv = 1.0 / jnp.sqrt(jnp.float32(1.0 + 1e-5))
    for l in range(L):
        e = _elin(edge_attr, We[l], be[l].reshape(1, H))
        parts = _edge_pass(x, e, src, dst)
        x = _node(
            x, parts,
            W1[l], b1[l].reshape(1, H),
            W2[l], b2[l].reshape(1, H),
            (gamma[l] * inv).reshape(1, H), beta[l].reshape(1, H),
        )
    return x


def kernel(x_solvent, edge_index_solvent, edge_attr_solvent, x_solvent_batch,
           x_solute, edge_index_solute, edge_attr_solute, x_solute_batch,
           global_feat, num_graphs,
           sv_W1, sv_b1, sv_W2, sv_b2, sv_We, sv_be, sv_gamma, sv_beta,
           su_W1, su_b1, su_W2, su_b2, su_We, su_be, su_gamma, su_beta,
           fc_W, fc_b, out_W, out_b):
    h_sv = _run_graph(x_solvent, edge_index_solvent, edge_attr_solvent,
                      sv_W1, sv_b1, sv_W2, sv_b2, sv_We, sv_be, sv_gamma,
                      sv_beta)
    h_su = _run_graph(x_solute, edge_index_solute, edge_attr_solute,
                      su_W1, su_b1, su_W2, su_b2, su_We, su_be, su_gamma,
                      su_beta)
    psv = _pool(h_sv, x_solvent_batch.astype(jnp.int32))
    psu = _pool(h_su, x_solute_batch.astype(jnp.int32))

    wo_p = jnp.pad(out_W, ((0, 0), (0, H - 1)))
    ob_p = jnp.pad(out_b.reshape(1, 1), ((0, 0), (0, H - 1)))
    out_full = _head(psv, psu, global_feat,
                     fc_W[:H], fc_W[H:2 * H], fc_W[2 * H:],
                     fc_b.reshape(1, H), wo_p, ob_p)
    return out_full[:, :1]


# speedup vs baseline: 2.7574x; 1.3792x over previous
import functools

import jax
import jax.numpy as jnp
from jax import lax
from jax.experimental import pallas as pl
from jax.experimental.pallas import tpu as pltpu
from jax.experimental.pallas import tpu_sc as plsc

N = 10000
E = 320000
D = 128
DE = 16
H = 128
G = 128
L = 3

NC = 2
NS = 16
NW = NC * NS

CHUNK = 80
NCHUNKS = E // CHUNK
CH_PER_TILE = NCHUNKS // NW
NPAIRS = (CH_PER_TILE - 1) // 2

ZCH = 80
NZCH = N // ZCH
ZCH_PER_TILE = -(-NZCH // NS)

_SC_MESH = plsc.VectorSubcoreMesh(core_axis_name="c", subcore_axis_name="s")


@functools.partial(
    pl.kernel,
    out_type=jax.ShapeDtypeStruct((NC, N, D), jnp.float32),
    mesh=_SC_MESH,
    scratch_types=[
        pltpu.VMEM((CHUNK,), jnp.int32),
        pltpu.VMEM((CHUNK,), jnp.int32),
        pltpu.VMEM((CHUNK,), jnp.int32),
        pltpu.VMEM((CHUNK,), jnp.int32),
        pltpu.VMEM((CHUNK, D), jnp.float32),
        pltpu.VMEM((CHUNK, D), jnp.float32),
        pltpu.VMEM((CHUNK, D), jnp.float32),
        pltpu.VMEM((CHUNK, D), jnp.float32),
        pltpu.VMEM_SHARED((N, D), jnp.float32),
        pltpu.SemaphoreType.DMA,
        pltpu.SemaphoreType.DMA,
        pltpu.SemaphoreType.DMA,
        pltpu.SemaphoreType.DMA,
    ],
)
def _edge_pass(x_hbm, e_hbm, src_hbm, dst_hbm, out_hbm,
               sidx0, sidx1, didx0, didx1, ev0, ev1, xv0, xv1, agg,
               semA0, semA1, semG0, semG1):
    cid = lax.axis_index("c")
    sid = lax.axis_index("s")
    wid = sid * NC + cid

    sidx = (sidx0, sidx1)
    didx = (didx0, didx1)
    ev = (ev0, ev1)
    xv = (xv0, xv1)
    semA = (semA0, semA1)
    semG = (semG0, semG1)

    def _zrow(r, carry):
        for c in range(D // 16):
            ev0[r, pl.ds(c * 16, 16)] = jnp.zeros((16,), jnp.float32)
        return carry

    lax.fori_loop(0, ZCH, _zrow, 0)
    for k in range(ZCH_PER_TILE):
        zc = k * NS + sid

        @pl.when(zc < NZCH)
        def _():
            pltpu.sync_copy(ev0.at[pl.ds(0, ZCH)], agg.at[pl.ds(zc * ZCH, ZCH)])

    plsc.subcore_barrier()

    def _start_a(c, b):
        base = (c * NW + wid) * CHUNK
        pltpu.async_copy(src_hbm.at[pl.ds(base, CHUNK)], sidx[b], semA[b])
        pltpu.async_copy(dst_hbm.at[pl.ds(base, CHUNK)], didx[b], semA[b])
        pltpu.async_copy(e_hbm.at[pl.ds(base, CHUNK)], ev[b], semA[b])

    def _wait_a(b):
        pltpu.make_async_copy(src_hbm.at[pl.ds(0, CHUNK)], sidx[b], semA[b]).wait()
        pltpu.make_async_copy(dst_hbm.at[pl.ds(0, CHUNK)], didx[b], semA[b]).wait()
        pltpu.make_async_copy(e_hbm.at[pl.ds(0, CHUNK)], ev[b], semA[b]).wait()

    def _start_g(b):
        pltpu.async_copy(x_hbm.at[sidx[b]], xv[b], semG[b])

    def _wait_g(b):
        pltpu.make_async_copy(x_hbm.at[sidx[b]], xv[b], semG[b]).wait()

    def _compute(b):
        xvb, evb = xv[b], ev[b]

        def _row(r, carry):
            for c in range(D // 16):
                s = pl.ds(c * 16, 16)
                xvb[r, s] = jnp.maximum(xvb[r, s] + evb[r, s], 0.0)
            return carry

        lax.fori_loop(0, CHUNK, _row, 0)

    _start_a(0, 0)
    _start_a(1, 1)
    _wait_a(0)
    _start_g(0)

    def _pair(j, carry):
        _wait_g(0)
        _wait_a(1)
        _start_g(1)
        _compute(0)
        pltpu.sync_copy(xv[0], agg.at[didx[0]], add=True)
        _start_a(2 * j + 2, 0)
        _wait_g(1)
        _wait_a(0)
        _start_g(0)
        _compute(1)
        pltpu.sync_copy(xv[1], agg.at[didx[1]], add=True)

        @pl.when(j < NPAIRS - 1)
        def _():
            _start_a(2 * j + 3, 1)

        return carry

    lax.fori_loop(0, NPAIRS, _pair, 0)

    _wait_g(0)
    _compute(0)
    pltpu.sync_copy(xv[0], agg.at[didx[0]], add=True)

    plsc.subcore_barrier()

    for k in range(ZCH_PER_TILE):
        zc = k * NS + sid

        @pl.when(zc < NZCH)
        def _():
            r0 = zc * ZCH
            pltpu.sync_copy(agg.at[pl.ds(r0, ZCH)], out_hbm.at[cid, pl.ds(r0, ZCH)])


PN = 2 * N
PG = 2 * G
PCH = 80
PNCH = PN // PCH
PCH_PER_TILE = -(-PNCH // NW)
G_PER_TILE = PG // NS


@functools.partial(
    pl.kernel,
    out_type=jax.ShapeDtypeStruct((NC, PG, D), jnp.float32),
    mesh=_SC_MESH,
    scratch_types=[
        pltpu.VMEM((PCH,), jnp.int32),
        pltpu.VMEM((PCH, D), jnp.float32),
        pltpu.VMEM_SHARED((PG, D), jnp.float32),
    ],
)
def _pool(h_hbm, b_hbm, out_hbm, bidx, hv, acc):
    cid = lax.axis_index("c")
    sid = lax.axis_index("s")
    wid = sid * NC + cid

    def _zrow(r, carry):
        for c in range(D // 16):
            hv[r, pl.ds(c * 16, 16)] = jnp.zeros((16,), jnp.float32)
        return carry

    lax.fori_loop(0, G_PER_TILE, _zrow, 0)
    pltpu.sync_copy(hv.at[pl.ds(0, G_PER_TILE)],
                    acc.at[pl.ds(sid * G_PER_TILE, G_PER_TILE)])
    plsc.subcore_barrier()

    def _chunk(i, carry):
        ch = i * NW + wid

        @pl.when(ch < PNCH)
        def _():
            base = ch * PCH
            pltpu.sync_copy(b_hbm.at[pl.ds(base, PCH)], bidx)
            pltpu.sync_copy(h_hbm.at[pl.ds(base, PCH)], hv)
            pltpu.sync_copy(hv, acc.at[bidx], add=True)

        return carry

    lax.fori_loop(0, PCH_PER_TILE, _chunk, 0)
    plsc.subcore_barrier()

    pltpu.sync_copy(acc.at[pl.ds(sid * G_PER_TILE, G_PER_TILE)],
                    out_hbm.at[cid, pl.ds(sid * G_PER_TILE, G_PER_TILE)])


BE = 1280


def _elin_body(ea_ref, we_ref, be_ref, o_ref):
    o_ref[...] = (
        jnp.dot(ea_ref[...], we_ref[...], preferred_element_type=jnp.float32)
        + be_ref[...]
    )


_elin = pl.pallas_call(
    _elin_body,
    grid=(E // BE,),
    in_specs=[
        pl.BlockSpec((BE, DE), lambda i: (i, 0)),
        pl.BlockSpec((DE, H), lambda i: (0, 0)),
        pl.BlockSpec((1, H), lambda i: (0, 0)),
    ],
    out_specs=pl.BlockSpec((BE, H), lambda i: (i, 0)),
    out_shape=jax.ShapeDtypeStruct((E, H), jnp.float32),
)

BN = 1000


def _node_body(x_ref, ap_ref, w1_ref, b1_ref, w2_ref, b2_ref, sc_ref, bt_ref,
               o_ref):
    h = x_ref[...] + ap_ref[0] + ap_ref[1]
    t = jnp.maximum(
        jnp.dot(h, w1_ref[...], preferred_element_type=jnp.float32) + b1_ref[...],
        0.0,
    )
    u = jnp.dot(t, w2_ref[...], preferred_element_type=jnp.float32) + b2_ref[...]
    o_ref[...] = jnp.maximum(u * sc_ref[...] + bt_ref[...], 0.0)


_node = pl.pallas_call(
    _node_body,
    grid=(N // BN,),
    in_specs=[
        pl.BlockSpec((BN, H), lambda i: (i, 0)),
        pl.BlockSpec((NC, BN, H), lambda i: (0, i, 0)),
        pl.BlockSpec((H, H), lambda i: (0, 0)),
        pl.BlockSpec((1, H), lambda i: (0, 0)),
        pl.BlockSpec((H, H), lambda i: (0, 0)),
        pl.BlockSpec((1, H), lambda i: (0, 0)),
        pl.BlockSpec((1, H), lambda i: (0, 0)),
        pl.BlockSpec((1, H), lambda i: (0, 0)),
    ],
    out_specs=pl.BlockSpec((BN, H), lambda i: (i, 0)),
    out_shape=jax.ShapeDtypeStruct((N, H), jnp.float32),
)


def _head_body(p_ref, gf_ref, wa_ref, wb_ref, wg_ref, fb_ref, wo_ref, ob_ref,
               o_ref):
    p = p_ref[...]
    gsv = p[0, :G, :] + p[1, :G, :]
    gsu = p[0, G:, :] + p[1, G:, :]
    h = (
        jnp.dot(gsv, wa_ref[...], preferred_element_type=jnp.float32)
        + jnp.dot(gsu, wb_ref[...], preferred_element_type=jnp.float32)
        + jnp.dot(gf_ref[...], wg_ref[...], preferred_element_type=jnp.float32)
        + fb_ref[...]
    )
    h = jnp.maximum(h, 0.0)
    o_ref[...] = jnp.dot(h, wo_ref[...], preferred_element_type=jnp.float32) + ob_ref[...]


_head = pl.pallas_call(
    _head_body,
    out_shape=jax.ShapeDtypeStruct((G, H), jnp.float32),
)


def kernel(x_solvent, edge_index_solvent, edge_attr_solvent, x_solvent_batch,
           x_solute, edge_index_solute, edge_attr_solute, x_solute_batch,
           global_feat, num_graphs,
           sv_W1, sv_b1, sv_W2, sv_b2, sv_We, sv_be, sv_gamma, sv_beta,
           su_W1, su_b1, su_W2, su_b2, su_We, su_be, su_gamma, su_beta,
           fc_W, fc_b, out_W, out_b):
    inv = 1.0 / jnp.sqrt(jnp.float32(1.0 + 1e-5))
    src1, dst1 = edge_index_solvent[0], edge_index_solvent[1]
    src2, dst2 = edge_index_solute[0], edge_index_solute[1]

    e1 = [_elin(edge_attr_solvent, sv_We[l], sv_be[l].reshape(1, H))
          for l in range(L)]
    e2 = [_elin(edge_attr_solute, su_We[l], su_be[l].reshape(1, H))
          for l in range(L)]

    x1, x2 = x_solvent, x_solute
    for l in range(L):
        p1 = _edge_pass(x1, e1[l], src1, dst1)
        p2 = _edge_pass(x2, e2[l], src2, dst2)
        x1 = _node(x1, p1, sv_W1[l], sv_b1[l].reshape(1, H),
                   sv_W2[l], sv_b2[l].reshape(1, H),
                   (sv_gamma[l] * inv).reshape(1, H), sv_beta[l].reshape(1, H))
        x2 = _node(x2, p2, su_W1[l], su_b1[l].reshape(1, H),
                   su_W2[l], su_b2[l].reshape(1, H),
                   (su_gamma[l] * inv).reshape(1, H), su_beta[l].reshape(1, H))

    h_cat = jnp.concatenate([x1, x2], axis=0)
    b_cat = jnp.concatenate([x_solvent_batch.astype(jnp.int32),
                             x_solute_batch.astype(jnp.int32) + G], axis=0)
    pooled = _pool(h_cat, b_cat)

    wo_p = jnp.pad(out_W, ((0, 0), (0, H - 1)))
    ob_p = jnp.pad(out_b.reshape(1, 1), ((0, 0), (0, H - 1)))
    out_full = _head(pooled, global_feat,
                     fc_W[:H], fc_W[H:2 * H], fc_W[2 * H:],
                     fc_b.reshape(1, H), wo_p, ob_p)
    return out_full[:, :1]


# speedup vs baseline: 3.2160x; 1.1663x over previous
import functools

import jax
import jax.numpy as jnp
from jax import lax
from jax.experimental import pallas as pl
from jax.experimental.pallas import tpu as pltpu
from jax.experimental.pallas import tpu_sc as plsc

N = 10000
E = 320000
D = 128
DE = 16
H = 128
G = 128
L = 3

NC = 2
NS = 16
NW = NC * NS

CHUNK = 80
NCHUNKS = E // CHUNK
CH_PER_TILE = NCHUNKS // NW
NQUADS = (CH_PER_TILE - 1) // 4

ZCH = 80
NZCH = N // ZCH
ZCH_PER_TILE = -(-NZCH // NS)

_SC_MESH = plsc.VectorSubcoreMesh(core_axis_name="c", subcore_axis_name="s")


@functools.partial(
    pl.kernel,
    out_type=jax.ShapeDtypeStruct((NC, N, D), jnp.float32),
    mesh=_SC_MESH,
    scratch_types=[
        pltpu.VMEM((CHUNK,), jnp.int32),
        pltpu.VMEM((CHUNK,), jnp.int32),
        pltpu.VMEM((CHUNK,), jnp.int32),
        pltpu.VMEM((CHUNK,), jnp.int32),
        pltpu.VMEM((CHUNK,), jnp.int32),
        pltpu.VMEM((CHUNK,), jnp.int32),
        pltpu.VMEM((CHUNK, D), jnp.float32),
        pltpu.VMEM((CHUNK, D), jnp.float32),
        pltpu.VMEM((CHUNK, D), jnp.float32),
        pltpu.VMEM((CHUNK, D), jnp.float32),
        pltpu.VMEM_SHARED((N, D), jnp.float32),
        pltpu.SemaphoreType.DMA,
        pltpu.SemaphoreType.DMA,
        pltpu.SemaphoreType.DMA,
        pltpu.SemaphoreType.DMA,
        pltpu.SemaphoreType.DMA,
        pltpu.SemaphoreType.DMA,
        pltpu.SemaphoreType.DMA,
        pltpu.SemaphoreType.DMA,
        pltpu.SemaphoreType.DMA,
        pltpu.SemaphoreType.DMA,
    ],
)
def _edge_pass(x_hbm, e_hbm, src_hbm, dst_hbm, out_hbm,
               sidx0, sidx1, didx0, didx1, didx2, didx3, ev0, ev1, xv0, xv1,
               agg, semA0, semA1, semG0, semG1, semS0, semS1,
               semD0, semD1, semD2, semD3):
    cid = lax.axis_index("c")
    sid = lax.axis_index("s")
    wid = sid * NC + cid

    sidx = (sidx0, sidx1)
    didx = (didx0, didx1, didx2, didx3)
    ev = (ev0, ev1)
    xv = (xv0, xv1)
    semA = (semA0, semA1)
    semG = (semG0, semG1)
    semS = (semS0, semS1)
    semD = (semD0, semD1, semD2, semD3)

    def _zrow(r, carry):
        for c in range(D // 16):
            ev0[r, pl.ds(c * 16, 16)] = jnp.zeros((16,), jnp.float32)
        return carry

    lax.fori_loop(0, ZCH, _zrow, 0)
    for k in range(ZCH_PER_TILE):
        zc = k * NS + sid

        @pl.when(zc < NZCH)
        def _():
            pltpu.sync_copy(ev0.at[pl.ds(0, ZCH)], agg.at[pl.ds(zc * ZCH, ZCH)])

    plsc.subcore_barrier()

    def _start_a(c, b):
        base = (c * NW + wid) * CHUNK
        pltpu.async_copy(src_hbm.at[pl.ds(base, CHUNK)], sidx[b], semA[b])
        pltpu.async_copy(e_hbm.at[pl.ds(base, CHUNK)], ev[b], semA[b])

    def _wait_a(b):
        pltpu.make_async_copy(src_hbm.at[pl.ds(0, CHUNK)], sidx[b], semA[b]).wait()
        pltpu.make_async_copy(e_hbm.at[pl.ds(0, CHUNK)], ev[b], semA[b]).wait()

    def _start_b(c, d):
        base = (c * NW + wid) * CHUNK
        pltpu.async_copy(dst_hbm.at[pl.ds(base, CHUNK)], didx[d], semD[d])

    def _wait_d(d):
        pltpu.make_async_copy(dst_hbm.at[pl.ds(0, CHUNK)], didx[d], semD[d]).wait()

    def _start_g(b):
        pltpu.async_copy(x_hbm.at[sidx[b]], xv[b], semG[b])

    def _wait_g(b):
        pltpu.make_async_copy(x_hbm.at[sidx[b]], xv[b], semG[b]).wait()

    def _start_s(b, d):
        pltpu.async_copy(xv[b], agg.at[didx[d]], semS[b], add=True)

    def _wait_s(b, d):
        pltpu.make_async_copy(xv[b], agg.at[didx[d]], semS[b]).wait()

    def _compute(b):
        xvb, evb = xv[b], ev[b]

        @functools.partial(plsc.parallel_loop, 0, CHUNK, unroll=4)
        def _row(r):
            for c in range(D // 16):
                s = pl.ds(c * 16, 16)
                xvb[r, s] = jnp.maximum(xvb[r, s] + evb[r, s], 0.0)

    _start_a(0, 0)
    _start_a(1, 1)
    _start_b(0, 0)
    _start_b(1, 1)
    _start_b(2, 2)
    _wait_a(0)
    _start_g(0)

    _wait_g(0)
    _wait_a(1)
    _start_g(1)
    _start_b(3, 3)
    _compute(0)
    _wait_d(0)
    _start_s(0, 0)
    _start_a(2, 0)

    _wait_g(1)
    _wait_a(0)
    _wait_s(0, 0)
    _start_g(0)
    _start_b(4, 0)
    _compute(1)
    _wait_d(1)
    _start_s(1, 1)
    _start_a(3, 1)

    _wait_g(0)
    _wait_a(1)
    _wait_s(1, 1)
    _start_g(1)
    _start_b(5, 1)
    _compute(0)
    _wait_d(2)
    _start_s(0, 2)
    _start_a(4, 0)

    _wait_g(1)
    _wait_a(0)
    _wait_s(0, 2)
    _start_g(0)
    _start_b(6, 2)
    _compute(1)
    _wait_d(3)
    _start_s(1, 3)
    _start_a(5, 1)

    def _quad(j, carry):
        for r in range(4):
            i = 4 * j + r
            b = r % 2
            nb = 1 - b
            d = r
            _wait_g(b)
            _wait_a(nb)
            _wait_s(nb, (r + 3) % 4)
            _start_g(nb)

            @pl.when(i + 3 < CH_PER_TILE)
            def _():
                _start_b(i + 3, (r + 3) % 4)

            _compute(b)
            _wait_d(d)
            _start_s(b, d)

            @pl.when(i + 2 < CH_PER_TILE)
            def _():
                _start_a(i + 2, b)

        return carry

    lax.fori_loop(1, NQUADS, _quad, 0)

    _wait_g(0)
    _wait_s(1, 3)
    _compute(0)
    _wait_d(0)
    _start_s(0, 0)
    _wait_s(0, 0)

    plsc.subcore_barrier()

    for k in range(ZCH_PER_TILE):
        zc = k * NS + sid

        @pl.when(zc < NZCH)
        def _():
            r0 = zc * ZCH
            pltpu.sync_copy(agg.at[pl.ds(r0, ZCH)], out_hbm.at[cid, pl.ds(r0, ZCH)])


PN = 2 * N
PG = 2 * G
PCH = 80
PNCH = PN // PCH
PCH_PER_TILE = -(-PNCH // NW)
G_PER_TILE = PG // NS


@functools.partial(
    pl.kernel,
    out_type=jax.ShapeDtypeStruct((NC, PG, D), jnp.float32),
    mesh=_SC_MESH,
    scratch_types=[
        pltpu.VMEM((PCH,), jnp.int32),
        pltpu.VMEM((PCH, D), jnp.float32),
        pltpu.VMEM_SHARED((PG, D), jnp.float32),
    ],
)
def _pool(h_hbm, b_hbm, out_hbm, bidx, hv, acc):
    cid = lax.axis_index("c")
    sid = lax.axis_index("s")
    wid = sid * NC + cid

    def _zrow(r, carry):
        for c in range(D // 16):
            hv[r, pl.ds(c * 16, 16)] = jnp.zeros((16,), jnp.float32)
        return carry

    lax.fori_loop(0, G_PER_TILE, _zrow, 0)
    pltpu.sync_copy(hv.at[pl.ds(0, G_PER_TILE)],
                    acc.at[pl.ds(sid * G_PER_TILE, G_PER_TILE)])
    plsc.subcore_barrier()

    def _chunk(i, carry):
        ch = i * NW + wid

        @pl.when(ch < PNCH)
        def _():
            base = ch * PCH
            pltpu.sync_copy(b_hbm.at[pl.ds(base, PCH)], bidx)
            pltpu.sync_copy(h_hbm.at[pl.ds(base, PCH)], hv)
            pltpu.sync_copy(hv, acc.at[bidx], add=True)

        return carry

    lax.fori_loop(0, PCH_PER_TILE, _chunk, 0)
    plsc.subcore_barrier()

    pltpu.sync_copy(acc.at[pl.ds(sid * G_PER_TILE, G_PER_TILE)],
                    out_hbm.at[cid, pl.ds(sid * G_PER_TILE, G_PER_TILE)])


BE = 1280


def _elin_body(ea_ref, we_ref, be_ref, o_ref):
    o_ref[...] = (
        jnp.dot(ea_ref[...], we_ref[...], preferred_element_type=jnp.float32)
        + be_ref[...]
    )


_elin = pl.pallas_call(
    _elin_body,
    grid=(E // BE,),
    in_specs=[
        pl.BlockSpec((BE, DE), lambda i: (i, 0)),
        pl.BlockSpec((DE, H), lambda i: (0, 0)),
        pl.BlockSpec((1, H), lambda i: (0, 0)),
    ],
    out_specs=pl.BlockSpec((BE, H), lambda i: (i, 0)),
    out_shape=jax.ShapeDtypeStruct((E, H), jnp.float32),
)

BN = 1000


def _node_body(x_ref, ap_ref, w1_ref, b1_ref, w2_ref, b2_ref, sc_ref, bt_ref,
               o_ref):
    h = x_ref[...] + ap_ref[0] + ap_ref[1]
    t = jnp.maximum(
        jnp.dot(h, w1_ref[...], preferred_element_type=jnp.float32) + b1_ref[...],
        0.0,
    )
    u = jnp.dot(t, w2_ref[...], preferred_element_type=jnp.float32) + b2_ref[...]
    o_ref[...] = jnp.maximum(u * sc_ref[...] + bt_ref[...], 0.0)


_node = pl.pallas_call(
    _node_body,
    grid=(N // BN,),
    in_specs=[
        pl.BlockSpec((BN, H), lambda i: (i, 0)),
        pl.BlockSpec((NC, BN, H), lambda i: (0, i, 0)),
        pl.BlockSpec((H, H), lambda i: (0, 0)),
        pl.BlockSpec((1, H), lambda i: (0, 0)),
        pl.BlockSpec((H, H), lambda i: (0, 0)),
        pl.BlockSpec((1, H), lambda i: (0, 0)),
        pl.BlockSpec((1, H), lambda i: (0, 0)),
        pl.BlockSpec((1, H), lambda i: (0, 0)),
    ],
    out_specs=pl.BlockSpec((BN, H), lambda i: (i, 0)),
    out_shape=jax.ShapeDtypeStruct((N, H), jnp.float32),
)


def _head_body(p_ref, gf_ref, wa_ref, wb_ref, wg_ref, fb_ref, wo_ref, ob_ref,
               o_ref):
    p = p_ref[...]
    gsv = p[0, :G, :] + p[1, :G, :]
    gsu = p[0, G:, :] + p[1, G:, :]
    h = (
        jnp.dot(gsv, wa_ref[...], preferred_element_type=jnp.float32)
        + jnp.dot(gsu, wb_ref[...], preferred_element_type=jnp.float32)
        + jnp.dot(gf_ref[...], wg_ref[...], preferred_element_type=jnp.float32)
        + fb_ref[...]
    )
    h = jnp.maximum(h, 0.0)
    o_ref[...] = jnp.dot(h, wo_ref[...], preferred_element_type=jnp.float32) + ob_ref[...]


_head = pl.pallas_call(
    _head_body,
    out_shape=jax.ShapeDtypeStruct((G, H), jnp.float32),
)


def kernel(x_solvent, edge_index_solvent, edge_attr_solvent, x_solvent_batch,
           x_solute, edge_index_solute, edge_attr_solute, x_solute_batch,
           global_feat, num_graphs,
           sv_W1, sv_b1, sv_W2, sv_b2, sv_We, sv_be, sv_gamma, sv_beta,
           su_W1, su_b1, su_W2, su_b2, su_We, su_be, su_gamma, su_beta,
           fc_W, fc_b, out_W, out_b):
    inv = 1.0 / jnp.sqrt(jnp.float32(1.0 + 1e-5))
    src1, dst1 = edge_index_solvent[0], edge_index_solvent[1]
    src2, dst2 = edge_index_solute[0], edge_index_solute[1]

    e1 = [_elin(edge_attr_solvent, sv_We[l], sv_be[l].reshape(1, H))
          for l in range(L)]
    e2 = [_elin(edge_attr_solute, su_We[l], su_be[l].reshape(1, H))
          for l in range(L)]

    x1, x2 = x_solvent, x_solute
    for l in range(L):
        p1 = _edge_pass(x1, e1[l], src1, dst1)
        p2 = _edge_pass(x2, e2[l], src2, dst2)
        x1 = _node(x1, p1, sv_W1[l], sv_b1[l].reshape(1, H),
                   sv_W2[l], sv_b2[l].reshape(1, H),
                   (sv_gamma[l] * inv).reshape(1, H), sv_beta[l].reshape(1, H))
        x2 = _node(x2, p2, su_W1[l], su_b1[l].reshape(1, H),
                   su_W2[l], su_b2[l].reshape(1, H),
                   (su_gamma[l] * inv).reshape(1, H), su_beta[l].reshape(1, H))

    h_cat = jnp.concatenate([x1, x2], axis=0)
    b_cat = jnp.concatenate([x_solvent_batch.astype(jnp.int32),
                             x_solute_batch.astype(jnp.int32) + G], axis=0)
    pooled = _pool(h_cat, b_cat)

    wo_p = jnp.pad(out_W, ((0, 0), (0, H - 1)))
    ob_p = jnp.pad(out_b.reshape(1, 1), ((0, 0), (0, H - 1)))
    out_full = _head(pooled, global_feat,
                     fc_W[:H], fc_W[H:2 * H], fc_W[2 * H:],
                     fc_b.reshape(1, H), wo_p, ob_p)
    return out_full[:, :1]
